# baseline (speedup 1.0000x reference)
import jax
import jax.numpy as jnp
from jax.experimental import pallas as pl
from jax.experimental.pallas import tpu as pltpu

B = 8
N = 14
D = 512
NG = 2
CG = D // NG
RP = N // 2
TAU = 0.5 / (N * N)
BETA = 4.0
COEF = TAU * BETA / N


def _mask_body(x_ref, o_ref, stash_v, m_v, mi_v, ui_v, wj_v):
    g = pl.program_id(0)
    t = pl.program_id(1)

    @pl.when(t < RP)
    def _fetch_phase():
        first = jnp.full((B, CG), t == 0)
        m_run = jnp.where(first, jnp.full((B, CG), -jnp.inf, jnp.float32), m_v[g])
        mi_run = jnp.where(first, jnp.zeros((B, CG), jnp.int32), mi_v[g])
        for di in range(2):
            for j in range(N):
                v = x_ref[di, j]
                stash_v[g, 2 * t + di, j] = v
                p = jnp.full((B, CG), (t * 2 + di) * N + j, jnp.int32)
                pred = v > m_run
                m_run = jnp.where(pred, v, m_run)
                mi_run = jnp.where(pred, p, mi_run)
        m_v[g] = m_run
        mi_v[g] = mi_run

    @pl.when(t >= RP)
    def _emit_phase():
        rp = t - RP

        @pl.when(t == RP)
        def _build_tables():
            mi = mi_v[g]
            i_max = (mi // N).astype(jnp.float32)
            j_max = (mi % N).astype(jnp.float32)
            for k in range(N):
                ui_v[g, k] = TAU - COEF * jnp.abs(float(k) - i_max)
                wj_v[g, k] = COEF * jnp.abs(float(k) - j_max)

        for di in range(2):
            ui = ui_v[g, 2 * rp + di]
            for j in range(N):
                o_ref[di, j] = stash_v[g, 2 * rp + di, j] * jnp.maximum(
                    ui - wj_v[g, j], -TAU
                )


@jax.jit
def _mask_layer(inputs):
    xt = inputs.transpose(1, 2, 0, 3)
    out = pl.pallas_call(
        _mask_body,
        grid=(NG, 2 * RP),
        in_specs=[
            pl.BlockSpec(
                (2, N, B, CG),
                lambda g, t: (jnp.minimum(t, RP - 1), 0, 0, g),
            )
        ],
        out_specs=pl.BlockSpec(
            (2, N, B, CG),
            lambda g, t: (jnp.maximum(t - RP, 0), 0, 0, g),
        ),
        out_shape=jax.ShapeDtypeStruct((N, N, B, D), jnp.float32),
        scratch_shapes=[
            pltpu.VMEM((NG, N, N, B, CG), jnp.float32),
            pltpu.VMEM((NG, B, CG), jnp.float32),
            pltpu.VMEM((NG, B, CG), jnp.int32),
            pltpu.VMEM((NG, N, B, CG), jnp.float32),
            pltpu.VMEM((NG, N, B, CG), jnp.float32),
        ],
    )(xt)
    return out.transpose(2, 0, 1, 3)


def kernel(inputs):
    return _mask_layer(inputs)


# speedup vs baseline: 3.8342x; 3.8342x over previous
import jax
import jax.numpy as jnp
from jax.experimental import pallas as pl

B = 8
N = 14
D = 512
CB = 256
TAU = 0.5 / (N * N)
BETA = 4.0
COEF = TAU * BETA / N


def _mask_body(x_ref, o_ref):
    m = x_ref[0, 0]
    mi = jnp.zeros((B, CB), jnp.int32)
    for i in range(N):
        for j in range(N):
            if i == 0 and j == 0:
                continue
            v = x_ref[i, j]
            pred = v > m
            m = jnp.where(pred, v, m)
            mi = jnp.where(pred, jnp.full((B, CB), i * N + j, jnp.int32), mi)

    i_max = (mi // N).astype(jnp.float32)
    j_max = (mi % N).astype(jnp.float32)

    ui = [TAU - COEF * jnp.abs(float(i) - i_max) for i in range(N)]
    wj = [COEF * jnp.abs(float(j) - j_max) for j in range(N)]

    for i in range(N):
        for j in range(N):
            mask = jnp.maximum(ui[i] - wj[j], -TAU)
            o_ref[i, j] = x_ref[i, j] * mask


@jax.jit
def _mask_layer(inputs):
    xt = inputs.transpose(1, 2, 0, 3)
    out = pl.pallas_call(
        _mask_body,
        grid=(D // CB,),
        in_specs=[pl.BlockSpec((N, N, B, CB), lambda k: (0, 0, 0, k))],
        out_specs=pl.BlockSpec((N, N, B, CB), lambda k: (0, 0, 0, k)),
        out_shape=jax.ShapeDtypeStruct((N, N, B, D), jnp.float32),
    )(xt)
    return out.transpose(2, 0, 1, 3)


def kernel(inputs):
    return _mask_layer(inputs)
